# baseline (speedup 1.0000x reference)
import functools

import jax
import jax.numpy as jnp
from jax import lax
from jax.experimental import pallas as pl
from jax.experimental.pallas import tpu as pltpu
from jax.experimental.pallas import tpu_sc as plsc

N = 10000
NPAD = 10240
E = 320000
NC = 2
NS = 16
NW = NC * NS
K = 96
NCH = 106
EPWP = NCH * K
EPAD = NW * EPWP
RPS = NPAD // NS

_mesh = plsc.VectorSubcoreMesh(
    core_axis_name="c", subcore_axis_name="s", num_cores=NC, num_subcores=NS)


def _wid():
    return lax.axis_index("s") * NC + lax.axis_index("c")


def _sc_deg_body(dst_r, ones_r, zeros_r, out_r, didx_v, ones_v, acc, ssem):
    c = lax.axis_index("c")
    s = lax.axis_index("s")
    w = _wid()
    pltpu.sync_copy(zeros_r.at[pl.ds(s * RPS, RPS)], acc.at[pl.ds(s * RPS, RPS)])
    pltpu.sync_copy(ones_r, ones_v)
    pltpu.sync_copy(dst_r.at[w], didx_v)
    plsc.subcore_barrier()

    def scat(j):
        pltpu.async_copy(ones_v, acc.at[didx_v.at[j]], ssem, add=True)

    def swait(j):
        pltpu.make_async_copy(ones_v, acc.at[didx_v.at[j]], ssem).wait()

    scat(0)

    def group(g, carry):
        j = 2 * g
        scat(j + 1)
        swait(j)
        scat(j + 2)
        swait(j + 1)
        return carry

    lax.fori_loop(0, NCH // 2 - 1, group, 0)
    j = NCH - 2
    scat(j + 1)
    swait(j)
    swait(j + 1)
    plsc.subcore_barrier()
    pltpu.sync_copy(acc.at[pl.ds(s * RPS, RPS)], out_r.at[c, pl.ds(s * RPS, RPS)])


_sc_deg = pl.kernel(
    _sc_deg_body,
    out_type=jax.ShapeDtypeStruct((NC, NPAD, 128), jnp.float32),
    mesh=_mesh,
    scratch_types=[
        pltpu.VMEM((NCH, K), jnp.int32),
        pltpu.VMEM((K, 128), jnp.float32),
        pltpu.VMEM_SHARED((NPAD, 128), jnp.float32),
        pltpu.SemaphoreType.DMA,
    ],
)


def _sc_agg_body(src_r, dst_r, y_r, zeros_r, out_r,
                 sidx_v, didx_v, r0, r1, acc, ssem):
    c = lax.axis_index("c")
    s = lax.axis_index("s")
    w = _wid()
    pltpu.sync_copy(zeros_r.at[pl.ds(s * RPS, RPS)], acc.at[pl.ds(s * RPS, RPS)])
    pltpu.sync_copy(src_r.at[w, 0], sidx_v)
    pltpu.sync_copy(dst_r.at[w], didx_v)
    plsc.subcore_barrier()

    def scat(j, buf):
        pltpu.async_copy(buf, acc.at[didx_v.at[j]], ssem, add=True)

    def swait(j, buf):
        pltpu.make_async_copy(buf, acc.at[didx_v.at[j]], ssem).wait()

    def gath(j, buf):
        pltpu.sync_copy(y_r.at[sidx_v.at[pl.ds(j * K, K)]], buf)

    gath(0, r0)

    def group(g, carry):
        j = 2 * g
        scat(j, r0)
        gath(j + 1, r1)
        swait(j, r0)
        scat(j + 1, r1)
        gath(j + 2, r0)
        swait(j + 1, r1)
        return carry

    lax.fori_loop(0, NCH // 2 - 1, group, 0)
    j = NCH - 2
    scat(j, r0)
    gath(j + 1, r1)
    swait(j, r0)
    scat(j + 1, r1)
    swait(j + 1, r1)

    plsc.subcore_barrier()
    pltpu.sync_copy(acc.at[pl.ds(s * RPS, RPS)], out_r.at[c, pl.ds(s * RPS, RPS)])


def _make_sc_agg(feat):
    return pl.kernel(
        _sc_agg_body,
        out_type=jax.ShapeDtypeStruct((NC, NPAD, feat), jnp.float32),
        mesh=_mesh,
        scratch_types=[
            pltpu.VMEM((EPWP,), jnp.int32),
            pltpu.VMEM((NCH, K), jnp.int32),
            pltpu.VMEM((K, feat), jnp.float32),
            pltpu.VMEM((K, feat), jnp.float32),
            pltpu.VMEM_SHARED((NPAD, feat), jnp.float32),
            pltpu.SemaphoreType.DMA,
        ],
    )


_sc_agg_128 = _make_sc_agg(128)


BLK = 1000


def _tc_mm1_body(x_r, w_r, degp_r, y_r, dinv_r):
    deg = 1.0 + degp_r[0, :, 0:1] + degp_r[1, :, 0:1]
    dinv = lax.rsqrt(deg)
    y_r[...] = dinv * jnp.dot(x_r[...], w_r[...],
                              preferred_element_type=jnp.float32)
    dinv_r[...] = dinv


def _tc_mm1(x, w1t, degp):
    return pl.pallas_call(
        _tc_mm1_body,
        grid=(N // BLK,),
        in_specs=[
            pl.BlockSpec((BLK, 128), lambda i: (i, 0)),
            pl.BlockSpec((128, 128), lambda i: (0, 0)),
            pl.BlockSpec((NC, BLK, 128), lambda i: (0, i, 0)),
        ],
        out_specs=[
            pl.BlockSpec((BLK, 128), lambda i: (i, 0)),
            pl.BlockSpec((BLK, 1), lambda i: (i, 0)),
        ],
        out_shape=[
            jax.ShapeDtypeStruct((N, 128), jnp.float32),
            jax.ShapeDtypeStruct((N, 1), jnp.float32),
        ],
    )(x, w1t, degp)


def _tc_l2_body(p_r, y1_r, dinv_r, w_r, y2_r):
    dinv = dinv_r[...]
    h = jnp.maximum(dinv * (p_r[0] + p_r[1] + y1_r[...]), 0.0)
    y2_r[...] = dinv * jnp.dot(h, w_r[...], preferred_element_type=jnp.float32)


def _tc_l2(p, y1, dinv, w2t_pad):
    return pl.pallas_call(
        _tc_l2_body,
        grid=(N // BLK,),
        in_specs=[
            pl.BlockSpec((NC, BLK, 128), lambda i: (0, i, 0)),
            pl.BlockSpec((BLK, 128), lambda i: (i, 0)),
            pl.BlockSpec((BLK, 1), lambda i: (i, 0)),
            pl.BlockSpec((128, 128), lambda i: (0, 0)),
        ],
        out_specs=pl.BlockSpec((BLK, 128), lambda i: (i, 0)),
        out_shape=jax.ShapeDtypeStruct((N, 128), jnp.float32),
    )(p, y1, dinv, w2t_pad)


def _tc_final_body(q_r, y2_r, dinv_r, o_r):
    o_r[...] = (dinv_r[...] * (q_r[0] + q_r[1] + y2_r[...]))[:, :64]


def _tc_final(q, y2, dinv):
    return pl.pallas_call(
        _tc_final_body,
        grid=(N // BLK,),
        in_specs=[
            pl.BlockSpec((NC, BLK, 128), lambda i: (0, i, 0)),
            pl.BlockSpec((BLK, 128), lambda i: (i, 0)),
            pl.BlockSpec((BLK, 1), lambda i: (i, 0)),
        ],
        out_specs=pl.BlockSpec((BLK, 64), lambda i: (i, 0)),
        out_shape=jax.ShapeDtypeStruct((N, 64), jnp.float32),
    )(q, y2, dinv)


def kernel(x, edge_index, W1, W2):
    dpw = EPWP - E // NW
    dsrc = (jnp.arange(NW * dpw, dtype=jnp.int32) * 57 % N).reshape(NW, dpw)
    ddst = N + (jnp.arange(NW * dpw, dtype=jnp.int32) % (NPAD - N))
    ddst = ddst.reshape(NW, dpw)
    src = jnp.concatenate(
        [edge_index[0].reshape(NW, E // NW), dsrc], axis=1)
    dst = jnp.concatenate(
        [edge_index[1].reshape(NW, E // NW), ddst], axis=1)
    src = src.reshape(NW, 1, EPWP)
    dst = dst.reshape(NW, NCH, K)
    zeros = jnp.zeros((NPAD, 128), jnp.float32)
    ones = jnp.ones((K, 128), jnp.float32)
    w2t_pad = jnp.pad(W2.T, ((0, 0), (0, 64)))

    degp = _sc_deg(dst, ones, zeros)
    y1, dinv = _tc_mm1(x, W1.T, degp)
    p = _sc_agg_128(src, dst, y1, zeros)
    y2 = _tc_l2(p, y1, dinv, w2t_pad)
    q = _sc_agg_128(src, dst, y2, zeros)
    return _tc_final(q, y2, dinv)


# speedup vs baseline: 1.0640x; 1.0640x over previous
import functools

import jax
import jax.numpy as jnp
from jax import lax
from jax.experimental import pallas as pl
from jax.experimental.pallas import tpu as pltpu
from jax.experimental.pallas import tpu_sc as plsc

N = 10000
NPAD = 10240
E = 320000
NC = 2
NS = 16
NW = NC * NS
K = 96
NCH = 106
EPWP = NCH * K
EPAD = NW * EPWP
RPS = NPAD // NS

_mesh = plsc.VectorSubcoreMesh(
    core_axis_name="c", subcore_axis_name="s", num_cores=NC, num_subcores=NS)


def _wid():
    return lax.axis_index("s") * NC + lax.axis_index("c")


def _sc_deg_body(dst_r, ones_r, zeros_r, out_r, didx_v, ones_v, acc, ssem):
    c = lax.axis_index("c")
    s = lax.axis_index("s")
    w = _wid()
    pltpu.sync_copy(zeros_r.at[pl.ds(s * RPS, RPS)], acc.at[pl.ds(s * RPS, RPS)])
    pltpu.sync_copy(ones_r, ones_v)
    pltpu.sync_copy(dst_r.at[w], didx_v)
    plsc.subcore_barrier()

    def scat(j):
        pltpu.async_copy(ones_v, acc.at[didx_v.at[j]], ssem, add=True)

    def swait(j):
        pltpu.make_async_copy(ones_v, acc.at[didx_v.at[j]], ssem).wait()

    scat(0)

    def group(g, carry):
        j = 2 * g
        scat(j + 1)
        swait(j)
        scat(j + 2)
        swait(j + 1)
        return carry

    lax.fori_loop(0, NCH // 2 - 1, group, 0)
    j = NCH - 2
    scat(j + 1)
    swait(j)
    swait(j + 1)
    plsc.subcore_barrier()
    pltpu.sync_copy(acc.at[pl.ds(s * RPS, RPS)], out_r.at[c, pl.ds(s * RPS, RPS)])


_sc_deg = pl.kernel(
    _sc_deg_body,
    out_type=jax.ShapeDtypeStruct((NC, NPAD, 128), jnp.float32),
    mesh=_mesh,
    scratch_types=[
        pltpu.VMEM((NCH, K), jnp.int32),
        pltpu.VMEM((K, 128), jnp.float32),
        pltpu.VMEM_SHARED((NPAD, 128), jnp.float32),
        pltpu.SemaphoreType.DMA,
    ],
)


def _sc_agg_body(src_r, dst_r, y_r, zeros_r, out_r,
                 sidx_v, didx_v, r0, r1, acc, ssem):
    c = lax.axis_index("c")
    s = lax.axis_index("s")
    w = _wid()
    pltpu.sync_copy(zeros_r.at[pl.ds(s * RPS, RPS)], acc.at[pl.ds(s * RPS, RPS)])
    pltpu.sync_copy(src_r.at[w, 0], sidx_v)
    pltpu.sync_copy(dst_r.at[w], didx_v)
    plsc.subcore_barrier()

    def scat(j, buf):
        pltpu.async_copy(buf, acc.at[didx_v.at[j]], ssem, add=True)

    def swait(j, buf):
        pltpu.make_async_copy(buf, acc.at[didx_v.at[j]], ssem).wait()

    def gath(j, buf):
        pltpu.sync_copy(y_r.at[sidx_v.at[pl.ds(j * K, K)]], buf)

    gath(0, r0)

    def group(g, carry):
        j = 2 * g
        scat(j, r0)
        gath(j + 1, r1)
        swait(j, r0)
        scat(j + 1, r1)
        gath(j + 2, r0)
        swait(j + 1, r1)
        return carry

    lax.fori_loop(0, NCH // 2 - 1, group, 0)
    j = NCH - 2
    scat(j, r0)
    gath(j + 1, r1)
    swait(j, r0)
    scat(j + 1, r1)
    swait(j + 1, r1)

    plsc.subcore_barrier()
    pltpu.sync_copy(acc.at[pl.ds(s * RPS, RPS)], out_r.at[c, pl.ds(s * RPS, RPS)])


def _make_sc_agg(feat, tc_tiling=True):
    params = None
    if not tc_tiling:
        params = pltpu.CompilerParams(use_tc_tiling_on_sc=False)
    return pl.kernel(
        _sc_agg_body,
        out_type=jax.ShapeDtypeStruct((NC, NPAD, feat), jnp.float32),
        mesh=_mesh,
        compiler_params=params,
        scratch_types=[
            pltpu.VMEM((EPWP,), jnp.int32),
            pltpu.VMEM((NCH, K), jnp.int32),
            pltpu.VMEM((K, feat), jnp.float32),
            pltpu.VMEM((K, feat), jnp.float32),
            pltpu.VMEM_SHARED((NPAD, feat), jnp.float32),
            pltpu.SemaphoreType.DMA,
        ],
    )


_sc_agg_128 = _make_sc_agg(128)
_sc_agg_64 = _make_sc_agg(64, tc_tiling=False)


BLK = 1000


def _tc_mm1_body(x_r, w_r, degp_r, y_r, dinv_r):
    deg = 1.0 + degp_r[0, :, 0:1] + degp_r[1, :, 0:1]
    dinv = lax.rsqrt(deg)
    y_r[...] = dinv * jnp.dot(x_r[...], w_r[...],
                              preferred_element_type=jnp.float32)
    dinv_r[...] = dinv


def _tc_mm1(x, w1t, degp):
    return pl.pallas_call(
        _tc_mm1_body,
        grid=(N // BLK,),
        in_specs=[
            pl.BlockSpec((BLK, 128), lambda i: (i, 0)),
            pl.BlockSpec((128, 128), lambda i: (0, 0)),
            pl.BlockSpec((NC, BLK, 128), lambda i: (0, i, 0)),
        ],
        out_specs=[
            pl.BlockSpec((BLK, 128), lambda i: (i, 0)),
            pl.BlockSpec((BLK, 1), lambda i: (i, 0)),
        ],
        out_shape=[
            jax.ShapeDtypeStruct((N, 128), jnp.float32),
            jax.ShapeDtypeStruct((N, 1), jnp.float32),
        ],
    )(x, w1t, degp)


def _tc_l2_body(p_r, y1_r, dinv_r, w_r, y2_r):
    dinv = dinv_r[...]
    h = jnp.maximum(dinv * (p_r[0] + p_r[1] + y1_r[...]), 0.0)
    y2_r[...] = dinv * jnp.dot(h, w_r[...], preferred_element_type=jnp.float32)


def _tc_l2(p, y1, dinv, w2t):
    return pl.pallas_call(
        _tc_l2_body,
        grid=(N // BLK,),
        in_specs=[
            pl.BlockSpec((NC, BLK, 128), lambda i: (0, i, 0)),
            pl.BlockSpec((BLK, 128), lambda i: (i, 0)),
            pl.BlockSpec((BLK, 1), lambda i: (i, 0)),
            pl.BlockSpec((128, 64), lambda i: (0, 0)),
        ],
        out_specs=pl.BlockSpec((BLK, 64), lambda i: (i, 0)),
        out_shape=jax.ShapeDtypeStruct((N, 64), jnp.float32),
    )(p, y1, dinv, w2t)


def _tc_final_body(q_r, y2_r, dinv_r, o_r):
    o_r[...] = dinv_r[...] * (q_r[0] + q_r[1] + y2_r[...])


def _tc_final(q, y2, dinv):
    return pl.pallas_call(
        _tc_final_body,
        grid=(N // BLK,),
        in_specs=[
            pl.BlockSpec((NC, BLK, 64), lambda i: (0, i, 0)),
            pl.BlockSpec((BLK, 64), lambda i: (i, 0)),
            pl.BlockSpec((BLK, 1), lambda i: (i, 0)),
        ],
        out_specs=pl.BlockSpec((BLK, 64), lambda i: (i, 0)),
        out_shape=jax.ShapeDtypeStruct((N, 64), jnp.float32),
    )(q, y2, dinv)


def kernel(x, edge_index, W1, W2):
    dpw = EPWP - E // NW
    dsrc = (jnp.arange(NW * dpw, dtype=jnp.int32) * 57 % N).reshape(NW, dpw)
    ddst = N + (jnp.arange(NW * dpw, dtype=jnp.int32) % (NPAD - N))
    ddst = ddst.reshape(NW, dpw)
    src = jnp.concatenate(
        [edge_index[0].reshape(NW, E // NW), dsrc], axis=1)
    dst = jnp.concatenate(
        [edge_index[1].reshape(NW, E // NW), ddst], axis=1)
    src = src.reshape(NW, 1, EPWP)
    dst = dst.reshape(NW, NCH, K)
    zeros = jnp.zeros((NPAD, 128), jnp.float32)
    zeros64 = jnp.zeros((NPAD, 64), jnp.float32)
    ones = jnp.ones((K, 128), jnp.float32)

    degp = _sc_deg(dst, ones, zeros)
    y1, dinv = _tc_mm1(x, W1.T, degp)
    p = _sc_agg_128(src, dst, y1, zeros)
    y2 = _tc_l2(p, y1, dinv, W2.T)
    q = _sc_agg_64(src, dst, y2, zeros64)
    return _tc_final(q, y2, dinv)


# speedup vs baseline: 1.1187x; 1.0515x over previous
import functools

import jax
import jax.numpy as jnp
from jax import lax
from jax.experimental import pallas as pl
from jax.experimental.pallas import tpu as pltpu
from jax.experimental.pallas import tpu_sc as plsc

N = 10000
NPAD = 10240
E = 320000
NC = 2
NS = 16
NW = NC * NS
K = 96
NCH = 106
EPWP = NCH * K
EPAD = NW * EPWP
RPS = NPAD // NS

_mesh = plsc.VectorSubcoreMesh(
    core_axis_name="c", subcore_axis_name="s", num_cores=NC, num_subcores=NS)


def _wid():
    return lax.axis_index("s") * NC + lax.axis_index("c")


def _sc_deg_body(dst_r, ones_r, zeros_r, out_r, didx_v, ones_v, acc, ssem):
    c = lax.axis_index("c")
    s = lax.axis_index("s")
    w = _wid()
    pltpu.sync_copy(zeros_r.at[pl.ds(s * RPS, RPS)], acc.at[pl.ds(s * RPS, RPS)])
    pltpu.sync_copy(ones_r, ones_v)
    pltpu.sync_copy(dst_r.at[w], didx_v)
    plsc.subcore_barrier()

    def scat(j):
        pltpu.async_copy(ones_v, acc.at[didx_v.at[j]], ssem, add=True)

    def swait(j):
        pltpu.make_async_copy(ones_v, acc.at[didx_v.at[j]], ssem).wait()

    scat(0)

    def group(g, carry):
        j = 2 * g
        scat(j + 1)
        swait(j)
        scat(j + 2)
        swait(j + 1)
        return carry

    lax.fori_loop(0, NCH // 2 - 1, group, 0)
    j = NCH - 2
    scat(j + 1)
    swait(j)
    swait(j + 1)
    plsc.subcore_barrier()
    pltpu.sync_copy(acc.at[pl.ds(s * RPS, RPS)], out_r.at[c, pl.ds(s * RPS, RPS)])


_sc_deg = pl.kernel(
    _sc_deg_body,
    out_type=jax.ShapeDtypeStruct((NC, NPAD, 64), jnp.int32),
    mesh=_mesh,
    compiler_params=pltpu.CompilerParams(use_tc_tiling_on_sc=False),
    scratch_types=[
        pltpu.VMEM((NCH, K), jnp.int32),
        pltpu.VMEM((K, 64), jnp.int32),
        pltpu.VMEM_SHARED((NPAD, 64), jnp.int32),
        pltpu.SemaphoreType.DMA,
    ],
)


def _sc_agg_body(src_r, dst_r, y_r, zeros_r, out_r,
                 sidx_v, didx_v, r0, r1, acc, ssem):
    c = lax.axis_index("c")
    s = lax.axis_index("s")
    w = _wid()
    pltpu.sync_copy(zeros_r.at[pl.ds(s * RPS, RPS)], acc.at[pl.ds(s * RPS, RPS)])
    pltpu.sync_copy(src_r.at[w, 0], sidx_v)
    pltpu.sync_copy(dst_r.at[w], didx_v)
    plsc.subcore_barrier()

    def scat(j, buf):
        pltpu.async_copy(buf, acc.at[didx_v.at[j]], ssem, add=True)

    def swait(j, buf):
        pltpu.make_async_copy(buf, acc.at[didx_v.at[j]], ssem).wait()

    def gath(j, buf):
        pltpu.sync_copy(y_r.at[sidx_v.at[pl.ds(j * K, K)]], buf)

    gath(0, r0)

    def group(g, carry):
        j = 2 * g
        scat(j, r0)
        gath(j + 1, r1)
        swait(j, r0)
        scat(j + 1, r1)
        gath(j + 2, r0)
        swait(j + 1, r1)
        return carry

    lax.fori_loop(0, NCH // 2 - 1, group, 0)
    j = NCH - 2
    scat(j, r0)
    gath(j + 1, r1)
    swait(j, r0)
    scat(j + 1, r1)
    swait(j + 1, r1)

    plsc.subcore_barrier()
    pltpu.sync_copy(acc.at[pl.ds(s * RPS, RPS)], out_r.at[c, pl.ds(s * RPS, RPS)])


def _make_sc_agg(feat, tc_tiling=True):
    params = None
    if not tc_tiling:
        params = pltpu.CompilerParams(use_tc_tiling_on_sc=False)
    return pl.kernel(
        _sc_agg_body,
        out_type=jax.ShapeDtypeStruct((NC, NPAD, feat), jnp.float32),
        mesh=_mesh,
        compiler_params=params,
        scratch_types=[
            pltpu.VMEM((EPWP,), jnp.int32),
            pltpu.VMEM((NCH, K), jnp.int32),
            pltpu.VMEM((K, feat), jnp.float32),
            pltpu.VMEM((K, feat), jnp.float32),
            pltpu.VMEM_SHARED((NPAD, feat), jnp.float32),
            pltpu.SemaphoreType.DMA,
        ],
    )


_sc_agg_128 = _make_sc_agg(128)
_sc_agg_64 = _make_sc_agg(64, tc_tiling=False)


BLK = 1000


def _tc_mm1_body(x_r, w_r, degp_r, y_r, dinv_r):
    cnt = (degp_r[0, :, 0:1] + degp_r[1, :, 0:1]) & 0xFFFF
    deg = 1.0 + cnt.astype(jnp.float32)
    dinv = lax.rsqrt(deg)
    y_r[...] = dinv * jnp.dot(x_r[...], w_r[...],
                              preferred_element_type=jnp.float32)
    dinv_r[...] = dinv


def _tc_mm1(x, w1t, degp):
    return pl.pallas_call(
        _tc_mm1_body,
        grid=(N // BLK,),
        in_specs=[
            pl.BlockSpec((BLK, 128), lambda i: (i, 0)),
            pl.BlockSpec((128, 128), lambda i: (0, 0)),
            pl.BlockSpec((NC, BLK, 64), lambda i: (0, i, 0)),
        ],
        out_specs=[
            pl.BlockSpec((BLK, 128), lambda i: (i, 0)),
            pl.BlockSpec((BLK, 1), lambda i: (i, 0)),
        ],
        out_shape=[
            jax.ShapeDtypeStruct((N, 128), jnp.float32),
            jax.ShapeDtypeStruct((N, 1), jnp.float32),
        ],
    )(x, w1t, degp)


def _tc_l2_body(p_r, y1_r, dinv_r, w_r, y2_r):
    dinv = dinv_r[...]
    h = jnp.maximum(dinv * (p_r[0] + p_r[1] + y1_r[...]), 0.0)
    y2_r[...] = dinv * jnp.dot(h, w_r[...], preferred_element_type=jnp.float32)


def _tc_l2(p, y1, dinv, w2t):
    return pl.pallas_call(
        _tc_l2_body,
        grid=(N // BLK,),
        in_specs=[
            pl.BlockSpec((NC, BLK, 128), lambda i: (0, i, 0)),
            pl.BlockSpec((BLK, 128), lambda i: (i, 0)),
            pl.BlockSpec((BLK, 1), lambda i: (i, 0)),
            pl.BlockSpec((128, 64), lambda i: (0, 0)),
        ],
        out_specs=pl.BlockSpec((BLK, 64), lambda i: (i, 0)),
        out_shape=jax.ShapeDtypeStruct((N, 64), jnp.float32),
    )(p, y1, dinv, w2t)


def _tc_final_body(q_r, y2_r, dinv_r, o_r):
    o_r[...] = dinv_r[...] * (q_r[0] + q_r[1] + y2_r[...])


def _tc_final(q, y2, dinv):
    return pl.pallas_call(
        _tc_final_body,
        grid=(N // BLK,),
        in_specs=[
            pl.BlockSpec((NC, BLK, 64), lambda i: (0, i, 0)),
            pl.BlockSpec((BLK, 64), lambda i: (i, 0)),
            pl.BlockSpec((BLK, 1), lambda i: (i, 0)),
        ],
        out_specs=pl.BlockSpec((BLK, 64), lambda i: (i, 0)),
        out_shape=jax.ShapeDtypeStruct((N, 64), jnp.float32),
    )(q, y2, dinv)


def kernel(x, edge_index, W1, W2):
    dpw = EPWP - E // NW
    dsrc = (jnp.arange(NW * dpw, dtype=jnp.int32) * 57 % N).reshape(NW, dpw)
    ddst = N + (jnp.arange(NW * dpw, dtype=jnp.int32) % (NPAD - N))
    ddst = ddst.reshape(NW, dpw)
    src = jnp.concatenate(
        [edge_index[0].reshape(NW, E // NW), dsrc], axis=1)
    dst = jnp.concatenate(
        [edge_index[1].reshape(NW, E // NW), ddst], axis=1)
    src = src.reshape(NW, 1, EPWP)
    dst = dst.reshape(NW, NCH, K)
    zeros = jnp.zeros((NPAD, 128), jnp.float32)
    zeros64 = jnp.zeros((NPAD, 64), jnp.float32)
    zeros_i = jnp.zeros((NPAD, 64), jnp.int32)
    ones = jnp.full((K, 64), 0x00010001, jnp.int32)

    degp = _sc_deg(dst, ones, zeros_i)
    y1, dinv = _tc_mm1(x, W1.T, degp)
    p = _sc_agg_128(src, dst, y1, zeros)
    y2 = _tc_l2(p, y1, dinv, W2.T)
    q = _sc_agg_64(src, dst, y2, zeros64)
    return _tc_final(q, y2, dinv)


# speedup vs baseline: 1.1190x; 1.0002x over previous
import functools

import jax
import jax.numpy as jnp
from jax import lax
from jax.experimental import pallas as pl
from jax.experimental.pallas import tpu as pltpu
from jax.experimental.pallas import tpu_sc as plsc

N = 10000
NPAD = 10240
E = 320000
NC = 2
NS = 16
NW = NC * NS
K = 96
NCH = 106
EPWP = NCH * K
EPAD = NW * EPWP
RPS = NPAD // NS

_mesh = plsc.VectorSubcoreMesh(
    core_axis_name="c", subcore_axis_name="s", num_cores=NC, num_subcores=NS)


def _wid():
    return lax.axis_index("s") * NC + lax.axis_index("c")


def _sc_deg_body(dst_r, ones_r, zeros_r, out_r, didx_v, ones_v, acc, ssem):
    c = lax.axis_index("c")
    s = lax.axis_index("s")
    w = _wid()
    pltpu.sync_copy(zeros_r.at[pl.ds(s * RPS, RPS)], acc.at[pl.ds(s * RPS, RPS)])
    pltpu.sync_copy(ones_r, ones_v)
    pltpu.sync_copy(dst_r.at[w], didx_v)
    plsc.subcore_barrier()

    def scat(j):
        pltpu.async_copy(ones_v, acc.at[didx_v.at[j]], ssem, add=True)

    def swait(j):
        pltpu.make_async_copy(ones_v, acc.at[didx_v.at[j]], ssem).wait()

    scat(0)
    scat(1)
    scat(2)

    def group(g, carry):
        j = 2 * g
        scat(j + 3)
        swait(j)
        scat(j + 4)
        swait(j + 1)
        return carry

    lax.fori_loop(0, (NCH - 4) // 2, group, 0)
    scat(NCH - 1)
    for j in range(NCH - 4, NCH):
        swait(j)
    plsc.subcore_barrier()
    pltpu.sync_copy(acc.at[pl.ds(s * RPS, RPS)], out_r.at[c, pl.ds(s * RPS, RPS)])


_sc_deg = pl.kernel(
    _sc_deg_body,
    out_type=jax.ShapeDtypeStruct((NC, NPAD, 64), jnp.int32),
    mesh=_mesh,
    compiler_params=pltpu.CompilerParams(use_tc_tiling_on_sc=False),
    scratch_types=[
        pltpu.VMEM((NCH, K), jnp.int32),
        pltpu.VMEM((K, 64), jnp.int32),
        pltpu.VMEM_SHARED((NPAD, 64), jnp.int32),
        pltpu.SemaphoreType.DMA,
    ],
)


def _sc_agg_body(src_r, dst_r, y_r, zeros_r, out_r,
                 sidx_v, didx_v, r0, r1, acc, ssem):
    c = lax.axis_index("c")
    s = lax.axis_index("s")
    w = _wid()
    pltpu.sync_copy(zeros_r.at[pl.ds(s * RPS, RPS)], acc.at[pl.ds(s * RPS, RPS)])
    pltpu.sync_copy(src_r.at[w, 0], sidx_v)
    pltpu.sync_copy(dst_r.at[w], didx_v)
    plsc.subcore_barrier()

    def scat(j, buf):
        pltpu.async_copy(buf, acc.at[didx_v.at[j]], ssem, add=True)

    def swait(j, buf):
        pltpu.make_async_copy(buf, acc.at[didx_v.at[j]], ssem).wait()

    def gath(j, buf):
        pltpu.sync_copy(y_r.at[sidx_v.at[pl.ds(j * K, K)]], buf)

    gath(0, r0)

    def group(g, carry):
        j = 2 * g
        scat(j, r0)
        gath(j + 1, r1)
        swait(j, r0)
        scat(j + 1, r1)
        gath(j + 2, r0)
        swait(j + 1, r1)
        return carry

    lax.fori_loop(0, NCH // 2 - 1, group, 0)
    j = NCH - 2
    scat(j, r0)
    gath(j + 1, r1)
    swait(j, r0)
    scat(j + 1, r1)
    swait(j + 1, r1)

    plsc.subcore_barrier()
    pltpu.sync_copy(acc.at[pl.ds(s * RPS, RPS)], out_r.at[c, pl.ds(s * RPS, RPS)])


def _make_sc_agg(feat, tc_tiling=True):
    params = None
    if not tc_tiling:
        params = pltpu.CompilerParams(use_tc_tiling_on_sc=False)
    return pl.kernel(
        _sc_agg_body,
        out_type=jax.ShapeDtypeStruct((NC, NPAD, feat), jnp.float32),
        mesh=_mesh,
        compiler_params=params,
        scratch_types=[
            pltpu.VMEM((EPWP,), jnp.int32),
            pltpu.VMEM((NCH, K), jnp.int32),
            pltpu.VMEM((K, feat), jnp.float32),
            pltpu.VMEM((K, feat), jnp.float32),
            pltpu.VMEM_SHARED((NPAD, feat), jnp.float32),
            pltpu.SemaphoreType.DMA,
        ],
    )


_sc_agg_128 = _make_sc_agg(128)
_sc_agg_64 = _make_sc_agg(64, tc_tiling=False)


BLK = 1000


def _tc_mm1_body(x_r, w_r, degp_r, y_r, dinv_r):
    cnt = (degp_r[0, :, 0:1] + degp_r[1, :, 0:1]) & 0xFFFF
    deg = 1.0 + cnt.astype(jnp.float32)
    dinv = lax.rsqrt(deg)
    y_r[...] = dinv * jnp.dot(x_r[...], w_r[...],
                              preferred_element_type=jnp.float32)
    dinv_r[...] = dinv


def _tc_mm1(x, w1t, degp):
    return pl.pallas_call(
        _tc_mm1_body,
        grid=(N // BLK,),
        in_specs=[
            pl.BlockSpec((BLK, 128), lambda i: (i, 0)),
            pl.BlockSpec((128, 128), lambda i: (0, 0)),
            pl.BlockSpec((NC, BLK, 64), lambda i: (0, i, 0)),
        ],
        out_specs=[
            pl.BlockSpec((BLK, 128), lambda i: (i, 0)),
            pl.BlockSpec((BLK, 1), lambda i: (i, 0)),
        ],
        out_shape=[
            jax.ShapeDtypeStruct((N, 128), jnp.float32),
            jax.ShapeDtypeStruct((N, 1), jnp.float32),
        ],
    )(x, w1t, degp)


def _tc_l2_body(p_r, y1_r, dinv_r, w_r, y2_r):
    dinv = dinv_r[...]
    h = jnp.maximum(dinv * (p_r[0] + p_r[1] + y1_r[...]), 0.0)
    y2_r[...] = dinv * jnp.dot(h, w_r[...], preferred_element_type=jnp.float32)


def _tc_l2(p, y1, dinv, w2t):
    return pl.pallas_call(
        _tc_l2_body,
        grid=(N // BLK,),
        in_specs=[
            pl.BlockSpec((NC, BLK, 128), lambda i: (0, i, 0)),
            pl.BlockSpec((BLK, 128), lambda i: (i, 0)),
            pl.BlockSpec((BLK, 1), lambda i: (i, 0)),
            pl.BlockSpec((128, 64), lambda i: (0, 0)),
        ],
        out_specs=pl.BlockSpec((BLK, 64), lambda i: (i, 0)),
        out_shape=jax.ShapeDtypeStruct((N, 64), jnp.float32),
    )(p, y1, dinv, w2t)


def _tc_final_body(q_r, y2_r, dinv_r, o_r):
    o_r[...] = dinv_r[...] * (q_r[0] + q_r[1] + y2_r[...])


def _tc_final(q, y2, dinv):
    return pl.pallas_call(
        _tc_final_body,
        grid=(N // BLK,),
        in_specs=[
            pl.BlockSpec((NC, BLK, 64), lambda i: (0, i, 0)),
            pl.BlockSpec((BLK, 64), lambda i: (i, 0)),
            pl.BlockSpec((BLK, 1), lambda i: (i, 0)),
        ],
        out_specs=pl.BlockSpec((BLK, 64), lambda i: (i, 0)),
        out_shape=jax.ShapeDtypeStruct((N, 64), jnp.float32),
    )(q, y2, dinv)


def kernel(x, edge_index, W1, W2):
    dpw = EPWP - E // NW
    dsrc = (jnp.arange(NW * dpw, dtype=jnp.int32) * 57 % N).reshape(NW, dpw)
    ddst = N + (jnp.arange(NW * dpw, dtype=jnp.int32) % (NPAD - N))
    ddst = ddst.reshape(NW, dpw)
    src = jnp.concatenate(
        [edge_index[0].reshape(NW, E // NW), dsrc], axis=1)
    dst = jnp.concatenate(
        [edge_index[1].reshape(NW, E // NW), ddst], axis=1)
    src = src.reshape(NW, 1, EPWP)
    dst = dst.reshape(NW, NCH, K)
    zeros = jnp.zeros((NPAD, 128), jnp.float32)
    zeros64 = jnp.zeros((NPAD, 64), jnp.float32)
    zeros_i = jnp.zeros((NPAD, 64), jnp.int32)
    ones = jnp.full((K, 64), 0x00010001, jnp.int32)

    degp = _sc_deg(dst, ones, zeros_i)
    y1, dinv = _tc_mm1(x, W1.T, degp)
    p = _sc_agg_128(src, dst, y1, zeros)
    y2 = _tc_l2(p, y1, dinv, W2.T)
    q = _sc_agg_64(src, dst, y2, zeros64)
    return _tc_final(q, y2, dinv)


# speedup vs baseline: 1.1215x; 1.0023x over previous
import functools

import jax
import jax.numpy as jnp
from jax import lax
from jax.experimental import pallas as pl
from jax.experimental.pallas import tpu as pltpu
from jax.experimental.pallas import tpu_sc as plsc

N = 10000
NPAD = 10240
E = 320000
NC = 2
NS = 16
NW = NC * NS
K = 96
NCH = 106
EPWP = NCH * K
EPAD = NW * EPWP
RPS = NPAD // NS

_mesh = plsc.VectorSubcoreMesh(
    core_axis_name="c", subcore_axis_name="s", num_cores=NC, num_subcores=NS)


def _wid():
    return lax.axis_index("s") * NC + lax.axis_index("c")


def _sc_deg_body(dst_r, ones_r, zeros_r, out_r, didx_v, ones_v, acc, ssem):
    c = lax.axis_index("c")
    s = lax.axis_index("s")
    w = _wid()
    pltpu.sync_copy(zeros_r.at[pl.ds(s * RPS, RPS)], acc.at[pl.ds(s * RPS, RPS)])
    pltpu.sync_copy(ones_r, ones_v)
    pltpu.sync_copy(dst_r.at[w], didx_v)
    plsc.subcore_barrier()

    def scat(j):
        pltpu.async_copy(ones_v, acc.at[didx_v.at[j]], ssem, add=True)

    def swait(j):
        pltpu.make_async_copy(ones_v, acc.at[didx_v.at[j]], ssem).wait()

    scat(0)
    scat(1)
    scat(2)

    def group(g, carry):
        j = 2 * g
        scat(j + 3)
        swait(j)
        scat(j + 4)
        swait(j + 1)
        return carry

    lax.fori_loop(0, (NCH - 4) // 2, group, 0)
    scat(NCH - 1)
    for j in range(NCH - 4, NCH):
        swait(j)
    plsc.subcore_barrier()
    pltpu.sync_copy(acc.at[pl.ds(s * RPS, RPS)], out_r.at[c, pl.ds(s * RPS, RPS)])


_sc_deg = pl.kernel(
    _sc_deg_body,
    out_type=jax.ShapeDtypeStruct((NC, NPAD, 64), jnp.int32),
    mesh=_mesh,
    compiler_params=pltpu.CompilerParams(use_tc_tiling_on_sc=False),
    scratch_types=[
        pltpu.VMEM((NCH, K), jnp.int32),
        pltpu.VMEM((K, 64), jnp.int32),
        pltpu.VMEM_SHARED((NPAD, 64), jnp.int32),
        pltpu.SemaphoreType.DMA,
    ],
)


def _sc_agg_body(src_r, dst_r, y_r, zeros_r, out_r,
                 sidx_v, didx_v, r0, r1, acc, ssem):
    c = lax.axis_index("c")
    s = lax.axis_index("s")
    w = _wid()
    pltpu.sync_copy(zeros_r.at[pl.ds(s * RPS, RPS)], acc.at[pl.ds(s * RPS, RPS)])
    pltpu.sync_copy(src_r.at[w, 0], sidx_v)
    pltpu.sync_copy(dst_r.at[w], didx_v)
    plsc.subcore_barrier()

    def scat(j, buf):
        pltpu.async_copy(buf, acc.at[didx_v.at[j]], ssem, add=True)

    def swait(j, buf):
        pltpu.make_async_copy(buf, acc.at[didx_v.at[j]], ssem).wait()

    def gath(j, buf):
        pltpu.sync_copy(y_r.at[sidx_v.at[pl.ds(j * K, K)]], buf)

    gath(0, r0)

    def group(g, carry):
        j = 2 * g
        scat(j, r0)
        gath(j + 1, r1)
        swait(j, r0)
        scat(j + 1, r1)
        gath(j + 2, r0)
        swait(j + 1, r1)
        return carry

    lax.fori_loop(0, NCH // 2 - 1, group, 0)
    j = NCH - 2
    scat(j, r0)
    gath(j + 1, r1)
    swait(j, r0)
    scat(j + 1, r1)
    swait(j + 1, r1)

    plsc.subcore_barrier()
    pltpu.sync_copy(acc.at[pl.ds(s * RPS, RPS)], out_r.at[c, pl.ds(s * RPS, RPS)])


def _make_sc_agg(feat, tc_tiling=True):
    params = None
    if not tc_tiling:
        params = pltpu.CompilerParams(use_tc_tiling_on_sc=False)
    return pl.kernel(
        _sc_agg_body,
        out_type=jax.ShapeDtypeStruct((NC, NPAD, feat), jnp.float32),
        mesh=_mesh,
        compiler_params=params,
        scratch_types=[
            pltpu.VMEM((EPWP,), jnp.int32),
            pltpu.VMEM((NCH, K), jnp.int32),
            pltpu.VMEM((K, feat), jnp.float32),
            pltpu.VMEM((K, feat), jnp.float32),
            pltpu.VMEM_SHARED((NPAD, feat), jnp.float32),
            pltpu.SemaphoreType.DMA,
        ],
    )


_sc_agg_128 = _make_sc_agg(128, tc_tiling=False)
_sc_agg_64 = _make_sc_agg(64, tc_tiling=False)


BLK = 1000


def _tc_mm1_body(x_r, w_r, degp_r, y_r, dinv_r):
    cnt = (degp_r[0, :, 0:1] + degp_r[1, :, 0:1]) & 0xFFFF
    deg = 1.0 + cnt.astype(jnp.float32)
    dinv = lax.rsqrt(deg)
    y_r[...] = dinv * jnp.dot(x_r[...], w_r[...],
                              preferred_element_type=jnp.float32)
    dinv_r[...] = dinv


def _tc_mm1(x, w1t, degp):
    return pl.pallas_call(
        _tc_mm1_body,
        grid=(N // BLK,),
        in_specs=[
            pl.BlockSpec((BLK, 128), lambda i: (i, 0)),
            pl.BlockSpec((128, 128), lambda i: (0, 0)),
            pl.BlockSpec((NC, BLK, 64), lambda i: (0, i, 0)),
        ],
        out_specs=[
            pl.BlockSpec((BLK, 128), lambda i: (i, 0)),
            pl.BlockSpec((BLK, 1), lambda i: (i, 0)),
        ],
        out_shape=[
            jax.ShapeDtypeStruct((N, 128), jnp.float32),
            jax.ShapeDtypeStruct((N, 1), jnp.float32),
        ],
    )(x, w1t, degp)


def _tc_l2_body(p_r, y1_r, dinv_r, w_r, y2_r):
    dinv = dinv_r[...]
    h = jnp.maximum(dinv * (p_r[0] + p_r[1] + y1_r[...]), 0.0)
    y2_r[...] = dinv * jnp.dot(h, w_r[...], preferred_element_type=jnp.float32)


def _tc_l2(p, y1, dinv, w2t):
    return pl.pallas_call(
        _tc_l2_body,
        grid=(N // BLK,),
        in_specs=[
            pl.BlockSpec((NC, BLK, 128), lambda i: (0, i, 0)),
            pl.BlockSpec((BLK, 128), lambda i: (i, 0)),
            pl.BlockSpec((BLK, 1), lambda i: (i, 0)),
            pl.BlockSpec((128, 64), lambda i: (0, 0)),
        ],
        out_specs=pl.BlockSpec((BLK, 64), lambda i: (i, 0)),
        out_shape=jax.ShapeDtypeStruct((N, 64), jnp.float32),
    )(p, y1, dinv, w2t)


def _tc_final_body(q_r, y2_r, dinv_r, o_r):
    o_r[...] = dinv_r[...] * (q_r[0] + q_r[1] + y2_r[...])


def _tc_final(q, y2, dinv):
    return pl.pallas_call(
        _tc_final_body,
        grid=(N // BLK,),
        in_specs=[
            pl.BlockSpec((NC, BLK, 64), lambda i: (0, i, 0)),
            pl.BlockSpec((BLK, 64), lambda i: (i, 0)),
            pl.BlockSpec((BLK, 1), lambda i: (i, 0)),
        ],
        out_specs=pl.BlockSpec((BLK, 64), lambda i: (i, 0)),
        out_shape=jax.ShapeDtypeStruct((N, 64), jnp.float32),
    )(q, y2, dinv)


def kernel(x, edge_index, W1, W2):
    dpw = EPWP - E // NW
    dsrc = (jnp.arange(NW * dpw, dtype=jnp.int32) * 57 % N).reshape(NW, dpw)
    ddst = N + (jnp.arange(NW * dpw, dtype=jnp.int32) % (NPAD - N))
    ddst = ddst.reshape(NW, dpw)
    src = jnp.concatenate(
        [edge_index[0].reshape(NW, E // NW), dsrc], axis=1)
    dst = jnp.concatenate(
        [edge_index[1].reshape(NW, E // NW), ddst], axis=1)
    src = src.reshape(NW, 1, EPWP)
    dst = dst.reshape(NW, NCH, K)
    zeros = jnp.zeros((NPAD, 128), jnp.float32)
    zeros64 = jnp.zeros((NPAD, 64), jnp.float32)
    zeros_i = jnp.zeros((NPAD, 64), jnp.int32)
    ones = jnp.full((K, 64), 0x00010001, jnp.int32)

    degp = _sc_deg(dst, ones, zeros_i)
    y1, dinv = _tc_mm1(x, W1.T, degp)
    p = _sc_agg_128(src, dst, y1, zeros)
    y2 = _tc_l2(p, y1, dinv, W2.T)
    q = _sc_agg_64(src, dst, y2, zeros64)
    return _tc_final(q, y2, dinv)


# speedup vs baseline: 1.1391x; 1.0157x over previous
import functools

import numpy as np

import jax
import jax.numpy as jnp
from jax import lax
from jax.experimental import pallas as pl
from jax.experimental.pallas import tpu as pltpu
from jax.experimental.pallas import tpu_sc as plsc

N = 10000
NPAD = 10240
E = 320000
NC = 2
NS = 16
NW = NC * NS
K = 96
NCH = 106
EPWP = NCH * K
EPAD = NW * EPWP
RPS = NPAD // NS

_mesh = plsc.VectorSubcoreMesh(
    core_axis_name="c", subcore_axis_name="s", num_cores=NC, num_subcores=NS)


def _wid():
    return lax.axis_index("s") * NC + lax.axis_index("c")


def _sc_deg_body(dst_r, ones_r, zeros_r, out_r, didx_v, ones_v, acc, ssem):
    c = lax.axis_index("c")
    s = lax.axis_index("s")
    w = _wid()
    pltpu.sync_copy(zeros_r.at[pl.ds(s * RPS, RPS)], acc.at[pl.ds(s * RPS, RPS)])
    pltpu.sync_copy(ones_r, ones_v)
    pltpu.sync_copy(dst_r.at[w], didx_v)
    plsc.subcore_barrier()

    def scat(j):
        pltpu.async_copy(ones_v, acc.at[didx_v.at[j]], ssem, add=True)

    def swait(j):
        pltpu.make_async_copy(ones_v, acc.at[didx_v.at[j]], ssem).wait()

    scat(0)
    scat(1)
    scat(2)

    def group(g, carry):
        j = 2 * g
        scat(j + 3)
        swait(j)
        scat(j + 4)
        swait(j + 1)
        return carry

    lax.fori_loop(0, (NCH - 4) // 2, group, 0)
    scat(NCH - 1)
    for j in range(NCH - 4, NCH):
        swait(j)
    plsc.subcore_barrier()
    pltpu.sync_copy(acc.at[pl.ds(s * RPS, RPS)], out_r.at[c, pl.ds(s * RPS, RPS)])


_sc_deg = pl.kernel(
    _sc_deg_body,
    out_type=jax.ShapeDtypeStruct((NC, NPAD, 64), jnp.int32),
    mesh=_mesh,
    compiler_params=pltpu.CompilerParams(use_tc_tiling_on_sc=False),
    scratch_types=[
        pltpu.VMEM((NCH, K), jnp.int32),
        pltpu.VMEM((K, 64), jnp.int32),
        pltpu.VMEM_SHARED((NPAD, 64), jnp.int32),
        pltpu.SemaphoreType.DMA,
    ],
)


def _sc_agg_body(src_r, dst_r, y_r, zeros_r, out_r,
                 sidx_v, didx_v, r0, r1, acc, ssem):
    c = lax.axis_index("c")
    s = lax.axis_index("s")
    w = _wid()
    pltpu.sync_copy(zeros_r.at[pl.ds(s * RPS, RPS)], acc.at[pl.ds(s * RPS, RPS)])
    pltpu.sync_copy(src_r.at[w, 0], sidx_v)
    pltpu.sync_copy(dst_r.at[w], didx_v)
    plsc.subcore_barrier()

    def scat(j, buf):
        pltpu.async_copy(buf, acc.at[didx_v.at[j]], ssem, add=True)

    def swait(j, buf):
        pltpu.make_async_copy(buf, acc.at[didx_v.at[j]], ssem).wait()

    def gath(j, buf):
        pltpu.sync_copy(y_r.at[sidx_v.at[pl.ds(j * K, K)]], buf)

    gath(0, r0)

    def group(g, carry):
        j = 2 * g
        scat(j, r0)
        gath(j + 1, r1)
        swait(j, r0)
        scat(j + 1, r1)
        gath(j + 2, r0)
        swait(j + 1, r1)
        return carry

    lax.fori_loop(0, NCH // 2 - 1, group, 0)
    j = NCH - 2
    scat(j, r0)
    gath(j + 1, r1)
    swait(j, r0)
    scat(j + 1, r1)
    swait(j + 1, r1)

    plsc.subcore_barrier()
    pltpu.sync_copy(acc.at[pl.ds(s * RPS, RPS)], out_r.at[c, pl.ds(s * RPS, RPS)])


def _make_sc_agg(feat, tc_tiling=True):
    params = None
    if not tc_tiling:
        params = pltpu.CompilerParams(use_tc_tiling_on_sc=False)
    return pl.kernel(
        _sc_agg_body,
        out_type=jax.ShapeDtypeStruct((NC, NPAD, feat), jnp.float32),
        mesh=_mesh,
        compiler_params=params,
        scratch_types=[
            pltpu.VMEM((EPWP,), jnp.int32),
            pltpu.VMEM((NCH, K), jnp.int32),
            pltpu.VMEM((K, feat), jnp.float32),
            pltpu.VMEM((K, feat), jnp.float32),
            pltpu.VMEM_SHARED((NPAD, feat), jnp.float32),
            pltpu.SemaphoreType.DMA,
        ],
    )


_sc_agg_128 = _make_sc_agg(128, tc_tiling=False)
_sc_agg_64 = _make_sc_agg(64, tc_tiling=False)


BLK = 2000


def _tc_mm1_body(x_r, w_r, degp_r, y_r, dinv_r):
    cnt = (degp_r[0, :, 0:1] + degp_r[1, :, 0:1]) & 0xFFFF
    deg = 1.0 + cnt.astype(jnp.float32)
    dinv = lax.rsqrt(deg)
    y_r[...] = dinv * jnp.dot(x_r[...], w_r[...],
                              preferred_element_type=jnp.float32)
    dinv_r[...] = dinv


def _tc_mm1(x, w1t, degp):
    return pl.pallas_call(
        _tc_mm1_body,
        grid=(N // BLK,),
        in_specs=[
            pl.BlockSpec((BLK, 128), lambda i: (i, 0)),
            pl.BlockSpec((128, 128), lambda i: (0, 0)),
            pl.BlockSpec((NC, BLK, 64), lambda i: (0, i, 0)),
        ],
        out_specs=[
            pl.BlockSpec((BLK, 128), lambda i: (i, 0)),
            pl.BlockSpec((BLK, 1), lambda i: (i, 0)),
        ],
        out_shape=[
            jax.ShapeDtypeStruct((N, 128), jnp.float32),
            jax.ShapeDtypeStruct((N, 1), jnp.float32),
        ],
    )(x, w1t, degp)


def _tc_l2_body(p_r, y1_r, dinv_r, w_r, y2_r):
    dinv = dinv_r[...]
    h = jnp.maximum(dinv * (p_r[0] + p_r[1] + y1_r[...]), 0.0)
    y2_r[...] = dinv * jnp.dot(h, w_r[...], preferred_element_type=jnp.float32)


def _tc_l2(p, y1, dinv, w2t):
    return pl.pallas_call(
        _tc_l2_body,
        grid=(N // BLK,),
        in_specs=[
            pl.BlockSpec((NC, BLK, 128), lambda i: (0, i, 0)),
            pl.BlockSpec((BLK, 128), lambda i: (i, 0)),
            pl.BlockSpec((BLK, 1), lambda i: (i, 0)),
            pl.BlockSpec((128, 64), lambda i: (0, 0)),
        ],
        out_specs=pl.BlockSpec((BLK, 64), lambda i: (i, 0)),
        out_shape=jax.ShapeDtypeStruct((N, 64), jnp.float32),
    )(p, y1, dinv, w2t)


def _tc_final_body(q_r, y2_r, dinv_r, o_r):
    o_r[...] = dinv_r[...] * (q_r[0] + q_r[1] + y2_r[...])


def _tc_final(q, y2, dinv):
    return pl.pallas_call(
        _tc_final_body,
        grid=(N // BLK,),
        in_specs=[
            pl.BlockSpec((NC, BLK, 64), lambda i: (0, i, 0)),
            pl.BlockSpec((BLK, 64), lambda i: (i, 0)),
            pl.BlockSpec((BLK, 1), lambda i: (i, 0)),
        ],
        out_specs=pl.BlockSpec((BLK, 64), lambda i: (i, 0)),
        out_shape=jax.ShapeDtypeStruct((N, 64), jnp.float32),
    )(q, y2, dinv)


def kernel(x, edge_index, W1, W2):
    dpw = EPWP - E // NW
    dsrc = jnp.asarray(
        (np.arange(NW * dpw, dtype=np.int32) * 57 % N).reshape(NW, dpw))
    ddst = jnp.asarray(
        (N + np.arange(NW * dpw, dtype=np.int32) % (NPAD - N)).reshape(NW, dpw)
        .astype(np.int32))
    src = jnp.concatenate(
        [edge_index[0].reshape(NW, E // NW), dsrc], axis=1)
    dst = jnp.concatenate(
        [edge_index[1].reshape(NW, E // NW), ddst], axis=1)
    src = src.reshape(NW, 1, EPWP)
    dst = dst.reshape(NW, NCH, K)
    zeros = jnp.zeros((NPAD, 128), jnp.float32)
    zeros64 = jnp.zeros((NPAD, 64), jnp.float32)
    zeros_i = jnp.zeros((NPAD, 64), jnp.int32)
    ones = jnp.full((K, 64), 0x00010001, jnp.int32)

    degp = _sc_deg(dst, ones, zeros_i)
    y1, dinv = _tc_mm1(x, W1.T, degp)
    p = _sc_agg_128(src, dst, y1, zeros)
    y2 = _tc_l2(p, y1, dinv, W2.T)
    q = _sc_agg_64(src, dst, y2, zeros64)
    return _tc_final(q, y2, dinv)


# speedup vs baseline: 1.1421x; 1.0026x over previous
import functools

import numpy as np

import jax
import jax.numpy as jnp
from jax import lax
from jax.experimental import pallas as pl
from jax.experimental.pallas import tpu as pltpu
from jax.experimental.pallas import tpu_sc as plsc

N = 10000
NPAD = 10240
E = 320000
NC = 2
NS = 16
NW = NC * NS
K = 96
NCH = 106
EPWP = NCH * K
EPAD = NW * EPWP
RPS = NPAD // NS

_mesh = plsc.VectorSubcoreMesh(
    core_axis_name="c", subcore_axis_name="s", num_cores=NC, num_subcores=NS)


def _wid():
    return lax.axis_index("s") * NC + lax.axis_index("c")


def _sc_deg_body(dst_r, ones_r, zeros_r, out_r, didx_v, ones_v, acc, ssem):
    c = lax.axis_index("c")
    s = lax.axis_index("s")
    w = _wid()
    pltpu.sync_copy(zeros_r.at[pl.ds(s * RPS, RPS)], acc.at[pl.ds(s * RPS, RPS)])
    pltpu.sync_copy(ones_r, ones_v)
    pltpu.sync_copy(dst_r.at[w], didx_v)
    plsc.subcore_barrier()

    def scat(j):
        pltpu.async_copy(ones_v, acc.at[didx_v.at[j]], ssem, add=True)

    def swait(j):
        pltpu.make_async_copy(ones_v, acc.at[didx_v.at[j]], ssem).wait()

    scat(0)
    scat(1)
    scat(2)

    def group(g, carry):
        j = 2 * g
        scat(j + 3)
        swait(j)
        scat(j + 4)
        swait(j + 1)
        return carry

    lax.fori_loop(0, (NCH - 4) // 2, group, 0)
    scat(NCH - 1)
    for j in range(NCH - 4, NCH):
        swait(j)
    plsc.subcore_barrier()
    pltpu.sync_copy(acc.at[pl.ds(s * RPS, RPS), pl.ds(0, 16)],
                    out_r.at[c, pl.ds(s * RPS, RPS)])


_sc_deg = pl.kernel(
    _sc_deg_body,
    out_type=jax.ShapeDtypeStruct((NC, NPAD, 16), jnp.int32),
    mesh=_mesh,
    compiler_params=pltpu.CompilerParams(use_tc_tiling_on_sc=False),
    scratch_types=[
        pltpu.VMEM((NCH, K), jnp.int32),
        pltpu.VMEM((K, 64), jnp.int32),
        pltpu.VMEM_SHARED((NPAD, 64), jnp.int32),
        pltpu.SemaphoreType.DMA,
    ],
)


def _sc_agg_body(src_r, dst_r, y_r, zeros_r, out_r,
                 sidx_v, didx_v, r0, r1, acc, ssem):
    c = lax.axis_index("c")
    s = lax.axis_index("s")
    w = _wid()
    pltpu.sync_copy(zeros_r.at[pl.ds(s * RPS, RPS)], acc.at[pl.ds(s * RPS, RPS)])
    pltpu.sync_copy(src_r.at[w, 0], sidx_v)
    pltpu.sync_copy(dst_r.at[w], didx_v)
    plsc.subcore_barrier()

    def scat(j, buf):
        pltpu.async_copy(buf, acc.at[didx_v.at[j]], ssem, add=True)

    def swait(j, buf):
        pltpu.make_async_copy(buf, acc.at[didx_v.at[j]], ssem).wait()

    def gath(j, buf):
        pltpu.sync_copy(y_r.at[sidx_v.at[pl.ds(j * K, K)]], buf)

    gath(0, r0)

    def group(g, carry):
        j = 2 * g
        scat(j, r0)
        gath(j + 1, r1)
        swait(j, r0)
        scat(j + 1, r1)
        gath(j + 2, r0)
        swait(j + 1, r1)
        return carry

    lax.fori_loop(0, NCH // 2 - 1, group, 0)
    j = NCH - 2
    scat(j, r0)
    gath(j + 1, r1)
    swait(j, r0)
    scat(j + 1, r1)
    swait(j + 1, r1)

    plsc.subcore_barrier()
    pltpu.sync_copy(acc.at[pl.ds(s * RPS, RPS)], out_r.at[c, pl.ds(s * RPS, RPS)])


def _make_sc_agg(feat, tc_tiling=True):
    params = None
    if not tc_tiling:
        params = pltpu.CompilerParams(use_tc_tiling_on_sc=False)
    return pl.kernel(
        _sc_agg_body,
        out_type=jax.ShapeDtypeStruct((NC, NPAD, feat), jnp.float32),
        mesh=_mesh,
        compiler_params=params,
        scratch_types=[
            pltpu.VMEM((EPWP,), jnp.int32),
            pltpu.VMEM((NCH, K), jnp.int32),
            pltpu.VMEM((K, feat), jnp.float32),
            pltpu.VMEM((K, feat), jnp.float32),
            pltpu.VMEM_SHARED((NPAD, feat), jnp.float32),
            pltpu.SemaphoreType.DMA,
        ],
    )


_sc_agg_128 = _make_sc_agg(128, tc_tiling=False)
_sc_agg_64 = _make_sc_agg(64, tc_tiling=False)


BLK = 2000


def _tc_mm1_body(x_r, w_r, degp_r, y_r, dinv_r):
    cnt = (degp_r[0, :, 0:1] + degp_r[1, :, 0:1]) & 0xFFFF
    deg = 1.0 + cnt.astype(jnp.float32)
    dinv = lax.rsqrt(deg)
    y_r[...] = dinv * jnp.dot(x_r[...], w_r[...],
                              preferred_element_type=jnp.float32)
    dinv_r[...] = dinv


def _tc_mm1(x, w1t, degp):
    return pl.pallas_call(
        _tc_mm1_body,
        grid=(N // BLK,),
        in_specs=[
            pl.BlockSpec((BLK, 128), lambda i: (i, 0)),
            pl.BlockSpec((128, 128), lambda i: (0, 0)),
            pl.BlockSpec((NC, BLK, 16), lambda i: (0, i, 0)),
        ],
        out_specs=[
            pl.BlockSpec((BLK, 128), lambda i: (i, 0)),
            pl.BlockSpec((BLK, 1), lambda i: (i, 0)),
        ],
        out_shape=[
            jax.ShapeDtypeStruct((N, 128), jnp.float32),
            jax.ShapeDtypeStruct((N, 1), jnp.float32),
        ],
    )(x, w1t, degp)


def _tc_l2_body(p_r, y1_r, dinv_r, w_r, y2_r):
    dinv = dinv_r[...]
    h = jnp.maximum(dinv * (p_r[0] + p_r[1] + y1_r[...]), 0.0)
    y2_r[...] = dinv * jnp.dot(h, w_r[...], preferred_element_type=jnp.float32)


def _tc_l2(p, y1, dinv, w2t):
    return pl.pallas_call(
        _tc_l2_body,
        grid=(N // BLK,),
        in_specs=[
            pl.BlockSpec((NC, BLK, 128), lambda i: (0, i, 0)),
            pl.BlockSpec((BLK, 128), lambda i: (i, 0)),
            pl.BlockSpec((BLK, 1), lambda i: (i, 0)),
            pl.BlockSpec((128, 64), lambda i: (0, 0)),
        ],
        out_specs=pl.BlockSpec((BLK, 64), lambda i: (i, 0)),
        out_shape=jax.ShapeDtypeStruct((N, 64), jnp.float32),
    )(p, y1, dinv, w2t)


def _tc_final_body(q_r, y2_r, dinv_r, o_r):
    o_r[...] = dinv_r[...] * (q_r[0] + q_r[1] + y2_r[...])


def _tc_final(q, y2, dinv):
    return pl.pallas_call(
        _tc_final_body,
        grid=(N // BLK,),
        in_specs=[
            pl.BlockSpec((NC, BLK, 64), lambda i: (0, i, 0)),
            pl.BlockSpec((BLK, 64), lambda i: (i, 0)),
            pl.BlockSpec((BLK, 1), lambda i: (i, 0)),
        ],
        out_specs=pl.BlockSpec((BLK, 64), lambda i: (i, 0)),
        out_shape=jax.ShapeDtypeStruct((N, 64), jnp.float32),
    )(q, y2, dinv)


def kernel(x, edge_index, W1, W2):
    dpw = EPWP - E // NW
    dsrc = jnp.asarray(
        (np.arange(NW * dpw, dtype=np.int32) * 57 % N).reshape(NW, dpw))
    ddst = jnp.asarray(
        (N + np.arange(NW * dpw, dtype=np.int32) % (NPAD - N)).reshape(NW, dpw)
        .astype(np.int32))
    src = jnp.concatenate(
        [edge_index[0].reshape(NW, E // NW), dsrc], axis=1)
    dst = jnp.concatenate(
        [edge_index[1].reshape(NW, E // NW), ddst], axis=1)
    src = src.reshape(NW, 1, EPWP)
    dst = dst.reshape(NW, NCH, K)
    zeros = jnp.zeros((NPAD, 128), jnp.float32)
    zeros64 = jnp.zeros((NPAD, 64), jnp.float32)
    zeros_i = jnp.zeros((NPAD, 64), jnp.int32)
    ones = jnp.full((K, 64), 0x00010001, jnp.int32)

    degp = _sc_deg(dst, ones, zeros_i)
    y1, dinv = _tc_mm1(x, W1.T, degp)
    p = _sc_agg_128(src, dst, y1, zeros)
    y2 = _tc_l2(p, y1, dinv, W2.T)
    q = _sc_agg_64(src, dst, y2, zeros64)
    return _tc_final(q, y2, dinv)
